# Initial kernel scaffold; baseline (speedup 1.0000x reference)
import functools

import jax
import jax.numpy as jnp
from jax import lax
from jax.experimental import pallas as pl
from jax.experimental.pallas import tpu as pltpu
from jax.experimental.pallas import tpu_sc as plsc

N = 10000
E = 320000
D = 128
DE = 16
DA = 144
NC = 2
NS = 16
NW = NC * NS
EPW = E // NW
CH = 80
NCHUNK = EPW // CH
NPW = N // NS
F32 = jnp.float32


_GBLK = 4000


def _g_body(ea_ref, we_ref, g_ref, s_ref):
    i = pl.program_id(0)
    g = jnp.dot(ea_ref[...], we_ref[...], preferred_element_type=F32)
    g_ref[...] = g

    @pl.when(i == 0)
    def _():
        s_ref[...] = jnp.zeros_like(s_ref)

    s_ref[...] += jnp.sum(g.reshape(-1, 8, D), axis=0)


def _edge_feats(ea, we):
    return pl.pallas_call(
        _g_body,
        grid=(E // _GBLK,),
        in_specs=[
            pl.BlockSpec((_GBLK, DE), lambda i: (i, 0)),
            pl.BlockSpec((DE, D), lambda i: (0, 0)),
        ],
        out_specs=[
            pl.BlockSpec((_GBLK, D), lambda i: (i, 0)),
            pl.BlockSpec((8, D), lambda i: (0, 0)),
        ],
        out_shape=[
            jax.ShapeDtypeStruct((E, D), F32),
            jax.ShapeDtypeStruct((8, D), F32),
        ],
    )(ea, we)


_RBLK = 2000


def _prep_body(x_ref, wl_ref, bl_ref, wr_ref, br_ref, xla_ref, xr_ref):
    x = x_ref[...]
    xl = jnp.dot(x, wl_ref[...], preferred_element_type=F32) + bl_ref[...]
    xr = jnp.dot(x, wr_ref[...], preferred_element_type=F32) + br_ref[...]
    xr_ref[...] = xr
    pad = jnp.where(lax.broadcasted_iota(jnp.int32, (1, DA - D), 1) == 0, 1.0, 0.0)
    xla_ref[...] = jnp.concatenate(
        [xl, jnp.broadcast_to(pad, (xl.shape[0], DA - D))], axis=1
    )


def _prep(x, wl, bl, wr, br):
    return pl.pallas_call(
        _prep_body,
        grid=(N // _RBLK,),
        in_specs=[
            pl.BlockSpec((_RBLK, D), lambda i: (i, 0)),
            pl.BlockSpec((D, D), lambda i: (0, 0)),
            pl.BlockSpec((1, D), lambda i: (0, 0)),
            pl.BlockSpec((D, D), lambda i: (0, 0)),
            pl.BlockSpec((1, D), lambda i: (0, 0)),
        ],
        out_specs=[
            pl.BlockSpec((_RBLK, DA), lambda i: (i, 0)),
            pl.BlockSpec((_RBLK, D), lambda i: (i, 0)),
        ],
        out_shape=[
            jax.ShapeDtypeStruct((N, DA), F32),
            jax.ShapeDtypeStruct((N, D), F32),
        ],
    )(x, wl, bl, wr, br)


def _combine_block(accs_ref, xla_ref, xr_ref, gsum_ref, att_ref, b_ref):
    acc = accs_ref[0] + accs_ref[1]
    xl = xla_ref[:, :D]
    xr = xr_ref[...]
    mean_g = jnp.sum(gsum_ref[...], axis=0, keepdims=True) * (1.0 / E)
    e = xl + xr + mean_g
    e = jnp.maximum(e, 0.2 * e)
    logit = jnp.sum(e * att_ref[...], axis=1, keepdims=True)
    w = jnp.exp(logit)
    colmask = jnp.where(
        lax.broadcasted_iota(jnp.int32, (1, DA - D), 1) == 0, 1.0, 0.0
    )
    den = jnp.sum(acc[:, D:] * colmask, axis=1, keepdims=True) + w
    num = acc[:, :D] + w * xl
    return jnp.maximum(num / (den + 1e-16) + b_ref[...], 0.0)


def _combine_body(accs_ref, xla_ref, xr_ref, gsum_ref, att_ref, b_ref, out_ref):
    out_ref[...] = _combine_block(accs_ref, xla_ref, xr_ref, gsum_ref, att_ref, b_ref)


def _combine_prep_body(accs_ref, xla_ref, xr_ref, gsum_ref, att_ref, b_ref,
                       wl_ref, bl_ref, wr_ref, br_ref, xla2_ref, xr2_ref):
    x = _combine_block(accs_ref, xla_ref, xr_ref, gsum_ref, att_ref, b_ref)
    xl = jnp.dot(x, wl_ref[...], preferred_element_type=F32) + bl_ref[...]
    xr = jnp.dot(x, wr_ref[...], preferred_element_type=F32) + br_ref[...]
    xr2_ref[...] = xr
    pad = jnp.where(lax.broadcasted_iota(jnp.int32, (1, DA - D), 1) == 0, 1.0, 0.0)
    xla2_ref[...] = jnp.concatenate(
        [xl, jnp.broadcast_to(pad, (xl.shape[0], DA - D))], axis=1
    )


def _mk_combine_specs():
    return [
        pl.BlockSpec((2, _RBLK, DA), lambda i: (0, i, 0)),
        pl.BlockSpec((_RBLK, DA), lambda i: (i, 0)),
        pl.BlockSpec((_RBLK, D), lambda i: (i, 0)),
        pl.BlockSpec((8, D), lambda i: (0, 0)),
        pl.BlockSpec((1, D), lambda i: (0, 0)),
        pl.BlockSpec((1, D), lambda i: (0, 0)),
    ]


def _combine(accs, xla, xr, gsum, att, b):
    return pl.pallas_call(
        _combine_body,
        grid=(N // _RBLK,),
        in_specs=_mk_combine_specs(),
        out_specs=pl.BlockSpec((_RBLK, D), lambda i: (i, 0)),
        out_shape=jax.ShapeDtypeStruct((N, D), F32),
    )(accs, xla, xr, gsum, att, b)


def _combine_prep(accs, xla, xr, gsum, att, b, wl, bl, wr, br):
    return pl.pallas_call(
        _combine_prep_body,
        grid=(N // _RBLK,),
        in_specs=_mk_combine_specs() + [
            pl.BlockSpec((D, D), lambda i: (0, 0)),
            pl.BlockSpec((1, D), lambda i: (0, 0)),
            pl.BlockSpec((D, D), lambda i: (0, 0)),
            pl.BlockSpec((1, D), lambda i: (0, 0)),
        ],
        out_specs=[
            pl.BlockSpec((_RBLK, DA), lambda i: (i, 0)),
            pl.BlockSpec((_RBLK, D), lambda i: (i, 0)),
        ],
        out_shape=[
            jax.ShapeDtypeStruct((N, DA), F32),
            jax.ShapeDtypeStruct((N, D), F32),
        ],
    )(accs, xla, xr, gsum, att, b, wl, bl, wr, br)


_SC_MESH = plsc.VectorSubcoreMesh(core_axis_name="c", subcore_axis_name="s")


@functools.partial(
    pl.kernel,
    out_type=jax.ShapeDtypeStruct((NC, N, DA), F32),
    mesh=_SC_MESH,
    scratch_types=[
        pltpu.VMEM((CH,), jnp.int32),
        pltpu.VMEM((CH,), jnp.int32),
        pltpu.VMEM((CH, DA), F32),
        pltpu.VMEM((CH, D), F32),
        pltpu.VMEM((CH, D), F32),
        pltpu.VMEM((CH, DA), F32),
        pltpu.VMEM((D,), F32),
        pltpu.VMEM_SHARED((N, DA), F32),
        pltpu.SemaphoreType.DMA,
        pltpu.SemaphoreType.DMA,
        pltpu.SemaphoreType.DMA,
    ],
)
def _edge_kernel(xla_hbm, xr_hbm, g_hbm, src_hbm, dst_hbm, att_hbm, out_hbm,
                 src_v, dst_v, xla_v, xr_v, g_v, msg_v, att_v, acc_sh,
                 sem1, sem2, sem3):
    c = lax.axis_index("c")
    s = lax.axis_index("s")
    wid = s * NC + c

    zero16 = jnp.zeros((16,), F32)

    def _zrow(i, _):
        for k in range(DA // 16):
            msg_v[i, pl.ds(k * 16, 16)] = zero16
        return 0

    lax.fori_loop(0, CH, _zrow, 0)

    row0 = s * NPW
    nfull = NPW // CH
    tail = NPW - nfull * CH
    for j in range(nfull):
        pltpu.sync_copy(msg_v, acc_sh.at[pl.ds(row0 + j * CH, CH)])
    pltpu.sync_copy(msg_v.at[pl.ds(0, tail)],
                    acc_sh.at[pl.ds(row0 + nfull * CH, tail)])

    pltpu.sync_copy(att_hbm, att_v)
    att_c = [att_v[pl.ds(k * 16, 16)] for k in range(D // 16)]
    onezero = jnp.where(lax.iota(jnp.int32, 16) == 0, 1.0, 0.0)

    plsc.subcore_barrier()

    def chunk_body(j, _):
        base = wid * EPW + j * CH
        pltpu.sync_copy(src_hbm.at[pl.ds(base, CH)], src_v)
        pltpu.sync_copy(dst_hbm.at[pl.ds(base, CH)], dst_v)
        cp1 = pltpu.async_copy(xla_hbm.at[src_v], xla_v, sem1)
        cp2 = pltpu.async_copy(xr_hbm.at[dst_v], xr_v, sem2)
        cp3 = pltpu.async_copy(g_hbm.at[pl.ds(base, CH)], g_v, sem3)
        cp1.wait()
        cp2.wait()
        cp3.wait()

        def edge_body(i, _):
            xlc = [xla_v[i, pl.ds(k * 16, 16)] for k in range(D // 16)]
            acc = zero16
            for k in range(D // 16):
                v = xlc[k] + xr_v[i, pl.ds(k * 16, 16)] + g_v[i, pl.ds(k * 16, 16)]
                v = jnp.maximum(v, 0.2 * v)
                acc = acc + v * att_c[k]
            logit = jnp.sum(acc)
            w = jnp.exp(jnp.full((16,), logit, F32))
            for k in range(D // 16):
                msg_v[i, pl.ds(k * 16, 16)] = xlc[k] * w
            msg_v[i, pl.ds(D, 16)] = w * onezero
            return 0

        lax.fori_loop(0, CH, edge_body, 0)
        pltpu.sync_copy(msg_v, acc_sh.at[dst_v], add=True)
        return 0

    lax.fori_loop(0, NCHUNK, chunk_body, 0)

    plsc.subcore_barrier()

    pltpu.sync_copy(acc_sh.at[pl.ds(row0, NPW)],
                    out_hbm.at[c, pl.ds(row0, NPW)])


def kernel(node_fts, edge_index, edge_attr, Wl1, bl1, Wr1, br1, We1, att1, b1,
           Wl2, bl2, Wr2, br2, We2, att2, b2):
    src = edge_index[0]
    dst = edge_index[1]
    bl1r = bl1.reshape(1, D)
    br1r = br1.reshape(1, D)
    bl2r = bl2.reshape(1, D)
    br2r = br2.reshape(1, D)
    att1r = att1.reshape(1, D)
    att2r = att2.reshape(1, D)
    b1r = b1.reshape(1, D)
    b2r = b2.reshape(1, D)

    g1, gsum1 = _edge_feats(edge_attr, We1)
    g2, gsum2 = _edge_feats(edge_attr, We2)

    xla1, xr1 = _prep(node_fts, Wl1, bl1r, Wr1, br1r)
    accs1 = _edge_kernel(xla1, xr1, g1, src, dst, att1)
    xla2, xr2 = _combine_prep(accs1, xla1, xr1, gsum1, att1r, b1r,
                              Wl2, bl2r, Wr2, br2r)
    accs2 = _edge_kernel(xla2, xr2, g2, src, dst, att2)
    return _combine(accs2, xla2, xr2, gsum2, att2r, b2r)


# speedup vs baseline: 13.4438x; 13.4438x over previous
import functools

import jax
import jax.numpy as jnp
from jax import lax
from jax.experimental import pallas as pl
from jax.experimental.pallas import tpu as pltpu
from jax.experimental.pallas import tpu_sc as plsc

N = 10000
E = 320000
D = 128
DE = 16
NC = 2
NS = 16
NW = NC * NS
EPW = E // NW
CH = 40
NCHUNK = EPW // CH
SBLK = 400
NSB = EPW // SBLK
KSB = SBLK // CH
KPAIR = KSB // 2
NR = N // 16
F32 = jnp.float32


_GBLK = 4000


def _g_body(ea_ref, we_ref, g_ref, s_ref):
    i = pl.program_id(0)
    g = jnp.dot(ea_ref[...], we_ref[...], preferred_element_type=F32)
    g_ref[...] = g

    @pl.when(i == 0)
    def _():
        s_ref[...] = jnp.zeros_like(s_ref)

    s_ref[...] += jnp.sum(g.reshape(-1, 8, D), axis=0)


def _edge_feats(ea, we):
    return pl.pallas_call(
        _g_body,
        grid=(E // _GBLK,),
        in_specs=[
            pl.BlockSpec((_GBLK, DE), lambda i: (i, 0)),
            pl.BlockSpec((DE, D), lambda i: (0, 0)),
        ],
        out_specs=[
            pl.BlockSpec((_GBLK, D), lambda i: (i, 0)),
            pl.BlockSpec((8, D), lambda i: (0, 0)),
        ],
        out_shape=[
            jax.ShapeDtypeStruct((E, D), F32),
            jax.ShapeDtypeStruct((8, D), F32),
        ],
    )(ea, we)


_RBLK = 2048
_NGRID = (N + _RBLK - 1) // _RBLK


def _prep_body(x_ref, wl_ref, bl_ref, wr_ref, br_ref, xl_ref, xr_ref):
    x = x_ref[...]
    xl_ref[...] = jnp.dot(x, wl_ref[...], preferred_element_type=F32) + bl_ref[...]
    xr_ref[...] = jnp.dot(x, wr_ref[...], preferred_element_type=F32) + br_ref[...]


def _prep(x, wl, bl, wr, br):
    return pl.pallas_call(
        _prep_body,
        grid=(_NGRID,),
        in_specs=[
            pl.BlockSpec((_RBLK, D), lambda i: (i, 0)),
            pl.BlockSpec((D, D), lambda i: (0, 0)),
            pl.BlockSpec((1, D), lambda i: (0, 0)),
            pl.BlockSpec((D, D), lambda i: (0, 0)),
            pl.BlockSpec((1, D), lambda i: (0, 0)),
        ],
        out_specs=[
            pl.BlockSpec((_RBLK, D), lambda i: (i, 0)),
            pl.BlockSpec((_RBLK, D), lambda i: (i, 0)),
        ],
        out_shape=[
            jax.ShapeDtypeStruct((N, D), F32),
            jax.ShapeDtypeStruct((N, D), F32),
        ],
    )(x, wl, bl, wr, br)


def _combine_block(accs_ref, den_ref, xl_ref, xr_ref, gsum_ref, att_ref, b_ref):
    acc = accs_ref[0] + accs_ref[1]
    xl = xl_ref[...]
    xr = xr_ref[...]
    den_e = lax.dot_general(
        den_ref[...], jnp.ones((NW, 1), F32),
        (((0,), (0,)), ((), ())), preferred_element_type=F32,
    )
    mean_g = jnp.sum(gsum_ref[...], axis=0, keepdims=True) * (1.0 / E)
    e = xl + xr + mean_g
    e = jnp.maximum(e, 0.2 * e)
    logit = jnp.sum(e * att_ref[...], axis=1, keepdims=True)
    w = jnp.exp(logit)
    den = den_e + w + 1e-16
    num = acc + w * xl
    return jnp.maximum(num / den + b_ref[...], 0.0)


def _combine_body(accs_ref, den_ref, xl_ref, xr_ref, gsum_ref, att_ref, b_ref,
                  out_ref):
    out_ref[...] = _combine_block(accs_ref, den_ref, xl_ref, xr_ref, gsum_ref,
                                  att_ref, b_ref)


def _combine_prep_body(accs_ref, den_ref, xl_ref, xr_ref, gsum_ref, att_ref,
                       b_ref, wl_ref, bl_ref, wr_ref, br_ref, xl2_ref, xr2_ref):
    x = _combine_block(accs_ref, den_ref, xl_ref, xr_ref, gsum_ref, att_ref,
                       b_ref)
    xl2_ref[...] = jnp.dot(x, wl_ref[...], preferred_element_type=F32) + bl_ref[...]
    xr2_ref[...] = jnp.dot(x, wr_ref[...], preferred_element_type=F32) + br_ref[...]


def _mk_combine_specs():
    return [
        pl.BlockSpec((2, _RBLK, D), lambda i: (0, i, 0)),
        pl.BlockSpec((NW, _RBLK), lambda i: (0, i)),
        pl.BlockSpec((_RBLK, D), lambda i: (i, 0)),
        pl.BlockSpec((_RBLK, D), lambda i: (i, 0)),
        pl.BlockSpec((8, D), lambda i: (0, 0)),
        pl.BlockSpec((1, D), lambda i: (0, 0)),
        pl.BlockSpec((1, D), lambda i: (0, 0)),
    ]


def _combine(accs, den, xl, xr, gsum, att, b):
    return pl.pallas_call(
        _combine_body,
        grid=(_NGRID,),
        in_specs=_mk_combine_specs(),
        out_specs=pl.BlockSpec((_RBLK, D), lambda i: (i, 0)),
        out_shape=jax.ShapeDtypeStruct((N, D), F32),
    )(accs, den, xl, xr, gsum, att, b)


def _combine_prep(accs, den, xl, xr, gsum, att, b, wl, bl, wr, br):
    return pl.pallas_call(
        _combine_prep_body,
        grid=(_NGRID,),
        in_specs=_mk_combine_specs() + [
            pl.BlockSpec((D, D), lambda i: (0, 0)),
            pl.BlockSpec((1, D), lambda i: (0, 0)),
            pl.BlockSpec((D, D), lambda i: (0, 0)),
            pl.BlockSpec((1, D), lambda i: (0, 0)),
        ],
        out_specs=[
            pl.BlockSpec((_RBLK, D), lambda i: (i, 0)),
            pl.BlockSpec((_RBLK, D), lambda i: (i, 0)),
        ],
        out_shape=[
            jax.ShapeDtypeStruct((N, D), F32),
            jax.ShapeDtypeStruct((N, D), F32),
        ],
    )(accs, den, xl, xr, gsum, att, b, wl, bl, wr, br)


_SC_MESH = plsc.VectorSubcoreMesh(core_axis_name="c", subcore_axis_name="s")


@functools.partial(
    pl.kernel,
    out_type=(
        jax.ShapeDtypeStruct((NC, N, D), F32),
        jax.ShapeDtypeStruct((NW * N,), F32),
    ),
    mesh=_SC_MESH,
    compiler_params=pltpu.CompilerParams(needs_layout_passes=False),
    scratch_types=[
        pltpu.VMEM((KSB + 1, CH), jnp.int32),
        pltpu.VMEM((KSB + 1, CH), jnp.int32),
        pltpu.VMEM((CH, D), F32),
        pltpu.VMEM((CH, D), F32),
        pltpu.VMEM((CH, D), F32),
        pltpu.VMEM((CH, D), F32),
        pltpu.VMEM((CH, D), F32),
        pltpu.VMEM((CH, D), F32),
        pltpu.VMEM((768,), F32),
        pltpu.VMEM((N,), F32),
        pltpu.VMEM((D,), F32),
        pltpu.VMEM_SHARED((N, D), F32),
        pltpu.SemaphoreType.DMA,
        pltpu.SemaphoreType.DMA,
        pltpu.SemaphoreType.DMA,
        pltpu.SemaphoreType.DMA,
        pltpu.SemaphoreType.DMA,
        pltpu.SemaphoreType.DMA,
        pltpu.SemaphoreType.DMA,
        pltpu.SemaphoreType.DMA,
        pltpu.SemaphoreType.DMA,
    ],
)
def _edge_kernel(xl_hbm, xr_hbm, g_hbm, src2_hbm, dst2_hbm, att_hbm,
                 acc_out, den_out,
                 sidx_v, didx_v,
                 xl0, xl1, xr0, xr1, g0, g1, w_v, den_v, att_v,
                 acc_sh, semxl0, semxl1, semxr0, semxr1, semg0, semg1,
                 sems0, sems1, semi):
    c = lax.axis_index("c")
    s = lax.axis_index("s")
    wid = s * NC + c
    ebase = wid * EPW
    rbase = wid * NCHUNK

    zero16 = jnp.zeros((16,), F32)
    xl = (xl0, xl1)
    xr = (xr0, xr1)
    g = (g0, g1)
    semxl = (semxl0, semxl1)
    semxr = (semxr0, semxr1)
    semg = (semg0, semg1)
    sems = (sems0, sems1)

    def _zden(i, _):
        den_v[pl.ds(i * 16, 16)] = zero16
        return 0

    lax.fori_loop(0, NR, _zden, 0)

    def _zrow(i, _):
        for k in range(D // 16):
            xl0[i, pl.ds(k * 16, 16)] = zero16
        return 0

    lax.fori_loop(0, CH, _zrow, 0)

    nz = N // CH
    ntrips = (nz - s + NS - 1) // NS

    def _zchunk(jj, _):
        j = s + jj * NS
        pltpu.sync_copy(xl0, acc_sh.at[pl.ds(j * CH, CH)])
        return 0

    lax.fori_loop(0, ntrips, _zchunk, 0)

    pltpu.sync_copy(att_hbm, att_v)
    att_c = [att_v[pl.ds(k * 16, 16)] for k in range(D // 16)]
    lane = lax.iota(jnp.int32, 16)
    tail_mask = lane < 8

    plsc.subcore_barrier()

    def _stage_idx(sb):
        widx = wid * NSB + sb
        pltpu.sync_copy(src2_hbm.at[widx], sidx_v.at[pl.ds(0, KSB)])
        pltpu.sync_copy(dst2_hbm.at[widx], didx_v.at[pl.ds(0, KSB)])

    def _issue_gathers(sb, q, b):
        base = ebase + sb * SBLK + q * CH
        pltpu.async_copy(xl_hbm.at[sidx_v.at[q]], xl[b], semxl[b])
        pltpu.async_copy(xr_hbm.at[didx_v.at[q]], xr[b], semxr[b])
        pltpu.async_copy(g_hbm.at[pl.ds(base, CH)], g[b], semg[b])

    def _wait_gathers(sb, q, b):
        base = ebase + sb * SBLK + q * CH
        pltpu.make_async_copy(xl_hbm.at[sidx_v.at[q]], xl[b], semxl[b]).wait()
        pltpu.make_async_copy(xr_hbm.at[didx_v.at[q]], xr[b], semxr[b]).wait()
        pltpu.make_async_copy(g_hbm.at[pl.ds(base, CH)], g[b], semg[b]).wait()

    def _compute(q, b):
        @plsc.parallel_loop(0, CH, unroll=8)
        def _(i):
            xlc = [xl[b][i, pl.ds(k * 16, 16)] for k in range(D // 16)]
            acc = zero16
            for k in range(D // 16):
                v = xlc[k] + xr[b][i, pl.ds(k * 16, 16)] + g[b][i, pl.ds(k * 16, 16)]
                v = jnp.maximum(v, 0.2 * v)
                acc = acc + v * att_c[k]
            logit = jnp.sum(acc)
            w = jnp.exp(jnp.full((16,), logit, F32))
            for k in range(D // 16):
                xl[b][i, pl.ds(k * 16, 16)] = xlc[k] * w
            w_v[pl.ds(i * 16, 16)] = w

        qv = jnp.full((16,), q, jnp.int32)
        for kq in range(3):
            mask = None if kq < 2 else tail_mask
            didx = plsc.load_gather(didx_v, [qv, kq * 16 + lane], mask=mask)
            wv = plsc.load_gather(w_v, [(kq * 16 + lane) * 16], mask=mask)
            plsc.addupdate_scatter(den_v, [didx], wv, mask=mask)

        pltpu.async_copy(xl[b], acc_sh.at[didx_v.at[q]], sems[b], add=True)

    def _wait_scatter(b):
        pltpu.make_async_copy(xl[b], acc_sh.at[didx_v.at[0]], sems[b]).wait()

    def sb_body(sb, _):
        def pair_body(t, _):
            q0 = 2 * t
            _wait_gathers(sb, q0, 0)

            @pl.when(t > 0)
            def _():
                _wait_scatter(1)

            _issue_gathers(sb, q0 + 1, 1)
            _compute(q0, 0)

            _wait_gathers(sb, q0 + 1, 1)

            @pl.when(t < KPAIR - 1)
            def _():
                _wait_scatter(0)
                _issue_gathers(sb, q0 + 2, 0)

            _compute(q0 + 1, 1)
            return 0

        lax.fori_loop(0, KPAIR, pair_body, 0)

        @pl.when(sb + 1 < NSB)
        def _():
            _wait_scatter(0)
            _wait_scatter(1)
            _stage_idx(sb + 1)
            _issue_gathers(sb + 1, 0, 0)

        return 0

    _stage_idx(0)
    _issue_gathers(0, 0, 0)
    lax.fori_loop(0, NSB, sb_body, 0)

    _wait_scatter(0)
    _wait_scatter(1)

    pltpu.sync_copy(den_v, den_out.at[pl.ds(wid * N, N)])

    plsc.subcore_barrier()

    def _wchunk(jj, _):
        j = s + jj * NS
        pltpu.sync_copy(acc_sh.at[pl.ds(j * CH, CH)],
                        acc_out.at[c, pl.ds(j * CH, CH)])
        return 0

    lax.fori_loop(0, ntrips, _wchunk, 0)


def kernel(node_fts, edge_index, edge_attr, Wl1, bl1, Wr1, br1, We1, att1, b1,
           Wl2, bl2, Wr2, br2, We2, att2, b2):
    src = edge_index[0]
    dst = edge_index[1]
    bl1r = bl1.reshape(1, D)
    br1r = br1.reshape(1, D)
    bl2r = bl2.reshape(1, D)
    br2r = br2.reshape(1, D)
    att1r = att1.reshape(1, D)
    att2r = att2.reshape(1, D)
    b1r = b1.reshape(1, D)
    b2r = b2.reshape(1, D)

    g1, gsum1 = _edge_feats(edge_attr, We1)
    g2, gsum2 = _edge_feats(edge_attr, We2)

    src2 = src.reshape(NW * NSB, KSB, CH)
    dst2 = dst.reshape(NW * NSB, KSB, CH)

    xl1, xr1 = _prep(node_fts, Wl1, bl1r, Wr1, br1r)
    accs1, den1 = _edge_kernel(xl1, xr1, g1, src2, dst2, att1)
    xl2, xr2 = _combine_prep(accs1, den1.reshape(NW, N), xl1, xr1, gsum1,
                             att1r, b1r, Wl2, bl2r, Wr2, br2r)
    accs2, den2 = _edge_kernel(xl2, xr2, g2, src2, dst2, att2)
    return _combine(accs2, den2.reshape(NW, N), xl2, xr2, gsum2, att2r, b2r)


# speedup vs baseline: 13.5270x; 1.0062x over previous
import functools

import jax
import jax.numpy as jnp
from jax import lax
from jax.experimental import pallas as pl
from jax.experimental.pallas import tpu as pltpu
from jax.experimental.pallas import tpu_sc as plsc

N = 10000
E = 320000
D = 128
DE = 16
NC = 2
NS = 16
NW = NC * NS
EPW = E // NW
CH = 40
NCHUNK = EPW // CH
SBLK = 400
NSB = EPW // SBLK
KSB = SBLK // CH
KPAIR = KSB // 2
NR = N // 16
F32 = jnp.float32


_GBLK2 = 2000
_E2 = E // 2


def _g_body(ea2_ref, wa_ref, wb_ref, ws_ref, gp_ref, s_ref):
    i = pl.program_id(0)
    ea2 = ea2_ref[...]
    a = jnp.dot(ea2, wa_ref[...], preferred_element_type=F32)
    b = jnp.dot(ea2, wb_ref[...], preferred_element_type=F32)
    gs = jnp.dot(ea2, ws_ref[...], preferred_element_type=F32)
    au = lax.bitcast_convert_type(a, jnp.uint32) + jnp.uint32(0x8000)
    bu = lax.bitcast_convert_type(b, jnp.uint32) + jnp.uint32(0x8000)
    word = (au >> jnp.uint32(16)) | (bu & jnp.uint32(0xFFFF0000))
    gp_ref[...] = lax.bitcast_convert_type(word, jnp.int32)

    @pl.when(i == 0)
    def _():
        s_ref[...] = jnp.zeros_like(s_ref)

    s_ref[...] += jnp.sum(gs.reshape(-1, 8, D), axis=0)


def _edge_feats(ea2, wa, wb, ws):
    return pl.pallas_call(
        _g_body,
        grid=(_E2 // _GBLK2,),
        in_specs=[
            pl.BlockSpec((_GBLK2, 2 * DE), lambda i: (i, 0)),
            pl.BlockSpec((2 * DE, D), lambda i: (0, 0)),
            pl.BlockSpec((2 * DE, D), lambda i: (0, 0)),
            pl.BlockSpec((2 * DE, D), lambda i: (0, 0)),
        ],
        out_specs=[
            pl.BlockSpec((_GBLK2, D), lambda i: (i, 0)),
            pl.BlockSpec((8, D), lambda i: (0, 0)),
        ],
        out_shape=[
            jax.ShapeDtypeStruct((_E2, D), jnp.int32),
            jax.ShapeDtypeStruct((8, D), F32),
        ],
    )(ea2, wa, wb, ws)


def _pack_weights(we):
    import numpy as np
    perm_a = np.array([32 * (w // 16) + (w % 16) for w in range(64)])
    wea = we[:, perm_a]
    web = we[:, perm_a + 16]
    z = jnp.zeros((DE, 64), F32)
    wa = jnp.concatenate(
        [jnp.concatenate([wea, z], 1), jnp.concatenate([z, wea], 1)], 0)
    wb = jnp.concatenate(
        [jnp.concatenate([web, z], 1), jnp.concatenate([z, web], 1)], 0)
    ws = jnp.concatenate([we, we], 0)
    return wa, wb, ws


_RBLK = 2048
_NGRID = (N + _RBLK - 1) // _RBLK


def _prep_body(x_ref, wl_ref, bl_ref, wr_ref, br_ref, xl_ref, xr_ref):
    x = x_ref[...]
    xl_ref[...] = jnp.dot(x, wl_ref[...], preferred_element_type=F32) + bl_ref[...]
    xr_ref[...] = jnp.dot(x, wr_ref[...], preferred_element_type=F32) + br_ref[...]


def _prep(x, wl, bl, wr, br):
    return pl.pallas_call(
        _prep_body,
        grid=(_NGRID,),
        in_specs=[
            pl.BlockSpec((_RBLK, D), lambda i: (i, 0)),
            pl.BlockSpec((D, D), lambda i: (0, 0)),
            pl.BlockSpec((1, D), lambda i: (0, 0)),
            pl.BlockSpec((D, D), lambda i: (0, 0)),
            pl.BlockSpec((1, D), lambda i: (0, 0)),
        ],
        out_specs=[
            pl.BlockSpec((_RBLK, D), lambda i: (i, 0)),
            pl.BlockSpec((_RBLK, D), lambda i: (i, 0)),
        ],
        out_shape=[
            jax.ShapeDtypeStruct((N, D), F32),
            jax.ShapeDtypeStruct((N, D), F32),
        ],
    )(x, wl, bl, wr, br)


def _combine_block(accs_ref, den_ref, xl_ref, xr_ref, gsum_ref, att_ref, b_ref):
    acc = accs_ref[0] + accs_ref[1]
    xl = xl_ref[...]
    xr = xr_ref[...]
    den_e = lax.dot_general(
        den_ref[...], jnp.ones((NW, 1), F32),
        (((0,), (0,)), ((), ())), preferred_element_type=F32,
    )
    mean_g = jnp.sum(gsum_ref[...], axis=0, keepdims=True) * (1.0 / E)
    e = xl + xr + mean_g
    e = jnp.maximum(e, 0.2 * e)
    logit = jnp.sum(e * att_ref[...], axis=1, keepdims=True)
    w = jnp.exp(logit)
    den = den_e + w + 1e-16
    num = acc + w * xl
    return jnp.maximum(num / den + b_ref[...], 0.0)


def _combine_body(accs_ref, den_ref, xl_ref, xr_ref, gsum_ref, att_ref, b_ref,
                  out_ref):
    out_ref[...] = _combine_block(accs_ref, den_ref, xl_ref, xr_ref, gsum_ref,
                                  att_ref, b_ref)


def _combine_prep_body(accs_ref, den_ref, xl_ref, xr_ref, gsum_ref, att_ref,
                       b_ref, wl_ref, bl_ref, wr_ref, br_ref, xl2_ref, xr2_ref):
    x = _combine_block(accs_ref, den_ref, xl_ref, xr_ref, gsum_ref, att_ref,
                       b_ref)
    xl2_ref[...] = jnp.dot(x, wl_ref[...], preferred_element_type=F32) + bl_ref[...]
    xr2_ref[...] = jnp.dot(x, wr_ref[...], preferred_element_type=F32) + br_ref[...]


def _mk_combine_specs():
    return [
        pl.BlockSpec((2, _RBLK, D), lambda i: (0, i, 0)),
        pl.BlockSpec((NW, _RBLK), lambda i: (0, i)),
        pl.BlockSpec((_RBLK, D), lambda i: (i, 0)),
        pl.BlockSpec((_RBLK, D), lambda i: (i, 0)),
        pl.BlockSpec((8, D), lambda i: (0, 0)),
        pl.BlockSpec((1, D), lambda i: (0, 0)),
        pl.BlockSpec((1, D), lambda i: (0, 0)),
    ]


def _combine(accs, den, xl, xr, gsum, att, b):
    return pl.pallas_call(
        _combine_body,
        grid=(_NGRID,),
        in_specs=_mk_combine_specs(),
        out_specs=pl.BlockSpec((_RBLK, D), lambda i: (i, 0)),
        out_shape=jax.ShapeDtypeStruct((N, D), F32),
    )(accs, den, xl, xr, gsum, att, b)


def _combine_prep(accs, den, xl, xr, gsum, att, b, wl, bl, wr, br):
    return pl.pallas_call(
        _combine_prep_body,
        grid=(_NGRID,),
        in_specs=_mk_combine_specs() + [
            pl.BlockSpec((D, D), lambda i: (0, 0)),
            pl.BlockSpec((1, D), lambda i: (0, 0)),
            pl.BlockSpec((D, D), lambda i: (0, 0)),
            pl.BlockSpec((1, D), lambda i: (0, 0)),
        ],
        out_specs=[
            pl.BlockSpec((_RBLK, D), lambda i: (i, 0)),
            pl.BlockSpec((_RBLK, D), lambda i: (i, 0)),
        ],
        out_shape=[
            jax.ShapeDtypeStruct((N, D), F32),
            jax.ShapeDtypeStruct((N, D), F32),
        ],
    )(accs, den, xl, xr, gsum, att, b, wl, bl, wr, br)


_SC_MESH = plsc.VectorSubcoreMesh(core_axis_name="c", subcore_axis_name="s")


@functools.partial(
    pl.kernel,
    out_type=(
        jax.ShapeDtypeStruct((NC, N, D), F32),
        jax.ShapeDtypeStruct((NW * N,), F32),
    ),
    mesh=_SC_MESH,
    compiler_params=pltpu.CompilerParams(needs_layout_passes=False),
    scratch_types=[
        pltpu.VMEM((KSB + 1, CH), jnp.int32),
        pltpu.VMEM((KSB + 1, CH), jnp.int32),
        pltpu.VMEM((CH, D), F32),
        pltpu.VMEM((CH, D), F32),
        pltpu.VMEM((CH, D), F32),
        pltpu.VMEM((CH, D), F32),
        pltpu.VMEM((CH * 64,), jnp.int32),
        pltpu.VMEM((CH * 64,), jnp.int32),
        pltpu.VMEM((768,), F32),
        pltpu.VMEM((N,), F32),
        pltpu.VMEM((D,), F32),
        pltpu.VMEM_SHARED((N, D), F32),
        pltpu.SemaphoreType.DMA,
        pltpu.SemaphoreType.DMA,
        pltpu.SemaphoreType.DMA,
        pltpu.SemaphoreType.DMA,
        pltpu.SemaphoreType.DMA,
        pltpu.SemaphoreType.DMA,
        pltpu.SemaphoreType.DMA,
        pltpu.SemaphoreType.DMA,
        pltpu.SemaphoreType.DMA,
    ],
)
def _edge_kernel(xl_hbm, xr_hbm, g_hbm, src2_hbm, dst2_hbm, att_hbm,
                 acc_out, den_out,
                 sidx_v, didx_v,
                 xl0, xl1, xr0, xr1, g0, g1, w_v, den_v, att_v,
                 acc_sh, semxl0, semxl1, semxr0, semxr1, semg0, semg1,
                 sems0, sems1, semi):
    c = lax.axis_index("c")
    s = lax.axis_index("s")
    wid = s * NC + c
    ebase = wid * EPW
    rbase = wid * NCHUNK

    zero16 = jnp.zeros((16,), F32)
    xl = (xl0, xl1)
    xr = (xr0, xr1)
    g = (g0, g1)
    semxl = (semxl0, semxl1)
    semxr = (semxr0, semxr1)
    semg = (semg0, semg1)
    sems = (sems0, sems1)

    def _zden(i, _):
        den_v[pl.ds(i * 16, 16)] = zero16
        return 0

    lax.fori_loop(0, NR, _zden, 0)

    def _zrow(i, _):
        for k in range(D // 16):
            xl0[i, pl.ds(k * 16, 16)] = zero16
        return 0

    lax.fori_loop(0, CH, _zrow, 0)

    nz = N // CH
    ntrips = (nz - s + NS - 1) // NS

    def _zchunk(jj, _):
        j = s + jj * NS
        pltpu.sync_copy(xl0, acc_sh.at[pl.ds(j * CH, CH)])
        return 0

    lax.fori_loop(0, ntrips, _zchunk, 0)

    pltpu.sync_copy(att_hbm, att_v)
    att_c = [att_v[pl.ds(k * 16, 16)] for k in range(D // 16)]
    lane = lax.iota(jnp.int32, 16)
    tail_mask = lane < 8

    plsc.subcore_barrier()

    def _stage_idx(sb):
        widx = wid * NSB + sb
        pltpu.sync_copy(src2_hbm.at[widx], sidx_v.at[pl.ds(0, KSB)])
        pltpu.sync_copy(dst2_hbm.at[widx], didx_v.at[pl.ds(0, KSB)])

    def _issue_gathers(sb, q, b):
        gbase = (ebase + sb * SBLK + q * CH) * 64
        pltpu.async_copy(xl_hbm.at[sidx_v.at[q]], xl[b], semxl[b])
        pltpu.async_copy(xr_hbm.at[didx_v.at[q]], xr[b], semxr[b])
        pltpu.async_copy(g_hbm.at[pl.ds(gbase, CH * 64)], g[b], semg[b])

    def _wait_gathers(sb, q, b):
        gbase = (ebase + sb * SBLK + q * CH) * 64
        pltpu.make_async_copy(xl_hbm.at[sidx_v.at[q]], xl[b], semxl[b]).wait()
        pltpu.make_async_copy(xr_hbm.at[didx_v.at[q]], xr[b], semxr[b]).wait()
        pltpu.make_async_copy(g_hbm.at[pl.ds(gbase, CH * 64)], g[b], semg[b]).wait()

    def _compute(q, b):
        @plsc.parallel_loop(0, CH, unroll=4)
        def _(i):
            xlc = [xl[b][i, pl.ds(k * 16, 16)] for k in range(D // 16)]
            acc = zero16
            for c in range(D // 32):
                gw = g[b][pl.ds(i * 64 + c * 16, 16)]
                glo = lax.bitcast_convert_type(lax.shift_left(gw, 16), F32)
                ghi = lax.bitcast_convert_type(
                    jnp.bitwise_and(gw, jnp.int32(-65536)), F32)
                for k, gg in ((2 * c, glo), (2 * c + 1, ghi)):
                    v = xlc[k] + xr[b][i, pl.ds(k * 16, 16)] + gg
                    v = jnp.maximum(v, 0.2 * v)
                    acc = acc + v * att_c[k]
            logit = jnp.sum(acc)
            w = jnp.exp(jnp.full((16,), logit, F32))
            for k in range(D // 16):
                xl[b][i, pl.ds(k * 16, 16)] = xlc[k] * w
            w_v[pl.ds(i * 16, 16)] = w

        qv = jnp.full((16,), q, jnp.int32)
        for kq in range(3):
            mask = None if kq < 2 else tail_mask
            didx = plsc.load_gather(didx_v, [qv, kq * 16 + lane], mask=mask)
            wv = plsc.load_gather(w_v, [(kq * 16 + lane) * 16], mask=mask)
            plsc.addupdate_scatter(den_v, [didx], wv, mask=mask)

        pltpu.async_copy(xl[b], acc_sh.at[didx_v.at[q]], sems[b], add=True)

    def _wait_scatter(b):
        pltpu.make_async_copy(xl[b], acc_sh.at[didx_v.at[0]], sems[b]).wait()

    def sb_body(sb, _):
        def pair_body(t, _):
            q0 = 2 * t
            _wait_gathers(sb, q0, 0)

            @pl.when(t > 0)
            def _():
                _wait_scatter(1)

            _issue_gathers(sb, q0 + 1, 1)
            _compute(q0, 0)

            _wait_gathers(sb, q0 + 1, 1)

            @pl.when(t < KPAIR - 1)
            def _():
                _wait_scatter(0)
                _issue_gathers(sb, q0 + 2, 0)

            _compute(q0 + 1, 1)
            return 0

        lax.fori_loop(0, KPAIR, pair_body, 0)

        @pl.when(sb + 1 < NSB)
        def _():
            _wait_scatter(0)
            _wait_scatter(1)
            _stage_idx(sb + 1)
            _issue_gathers(sb + 1, 0, 0)

        return 0

    _stage_idx(0)
    _issue_gathers(0, 0, 0)
    lax.fori_loop(0, NSB, sb_body, 0)

    _wait_scatter(0)
    _wait_scatter(1)

    pltpu.sync_copy(den_v, den_out.at[pl.ds(wid * N, N)])

    plsc.subcore_barrier()

    def _wchunk(jj, _):
        j = s + jj * NS
        pltpu.sync_copy(acc_sh.at[pl.ds(j * CH, CH)],
                        acc_out.at[c, pl.ds(j * CH, CH)])
        return 0

    lax.fori_loop(0, ntrips, _wchunk, 0)


def kernel(node_fts, edge_index, edge_attr, Wl1, bl1, Wr1, br1, We1, att1, b1,
           Wl2, bl2, Wr2, br2, We2, att2, b2):
    src = edge_index[0]
    dst = edge_index[1]
    bl1r = bl1.reshape(1, D)
    br1r = br1.reshape(1, D)
    bl2r = bl2.reshape(1, D)
    br2r = br2.reshape(1, D)
    att1r = att1.reshape(1, D)
    att2r = att2.reshape(1, D)
    b1r = b1.reshape(1, D)
    b2r = b2.reshape(1, D)

    ea2 = edge_attr.reshape(_E2, 2 * DE)
    wa1, wb1, ws1 = _pack_weights(We1)
    wa2, wb2, ws2 = _pack_weights(We2)
    g1, gsum1 = _edge_feats(ea2, wa1, wb1, ws1)
    g2, gsum2 = _edge_feats(ea2, wa2, wb2, ws2)
    g1 = g1.reshape(_E2 * D)
    g2 = g2.reshape(_E2 * D)

    src2 = src.reshape(NW * NSB, KSB, CH)
    dst2 = dst.reshape(NW * NSB, KSB, CH)

    xl1, xr1 = _prep(node_fts, Wl1, bl1r, Wr1, br1r)
    accs1, den1 = _edge_kernel(xl1, xr1, g1, src2, dst2, att1)
    xl2, xr2 = _combine_prep(accs1, den1.reshape(NW, N), xl1, xr1, gsum1,
                             att1r, b1r, Wl2, bl2r, Wr2, br2r)
    accs2, den2 = _edge_kernel(xl2, xr2, g2, src2, dst2, att2)
    return _combine(accs2, den2.reshape(NW, N), xl2, xr2, gsum2, att2r, b2r)


# speedup vs baseline: 13.5306x; 1.0003x over previous
import functools

import jax
import jax.numpy as jnp
from jax import lax
from jax.experimental import pallas as pl
from jax.experimental.pallas import tpu as pltpu
from jax.experimental.pallas import tpu_sc as plsc

N = 10000
E = 320000
D = 128
DE = 16
NC = 2
NS = 16
NW = NC * NS
EPW = E // NW
CH = 40
NCHUNK = EPW // CH
SBLK = 400
NSB = EPW // SBLK
KSB = SBLK // CH
KPAIR = KSB // 2
NR = N // 16
F32 = jnp.float32


_GBLK = 4000


def _g_body(ea_ref, we_ref, g_ref, s_ref):
    i = pl.program_id(0)
    g = jnp.dot(ea_ref[...], we_ref[...], preferred_element_type=F32)
    g_ref[...] = g

    @pl.when(i == 0)
    def _():
        s_ref[...] = jnp.zeros_like(s_ref)

    s_ref[...] += jnp.sum(g.reshape(-1, 8, D), axis=0)


def _edge_feats(ea, we):
    return pl.pallas_call(
        _g_body,
        grid=(E // _GBLK,),
        in_specs=[
            pl.BlockSpec((_GBLK, DE), lambda i: (i, 0)),
            pl.BlockSpec((DE, D), lambda i: (0, 0)),
        ],
        out_specs=[
            pl.BlockSpec((_GBLK, D), lambda i: (i, 0)),
            pl.BlockSpec((8, D), lambda i: (0, 0)),
        ],
        out_shape=[
            jax.ShapeDtypeStruct((E, D), F32),
            jax.ShapeDtypeStruct((8, D), F32),
        ],
    )(ea, we)


_RBLK = 2048
_NGRID = (N + _RBLK - 1) // _RBLK


def _prep_body(x_ref, wl_ref, bl_ref, wr_ref, br_ref, xl_ref, xr_ref):
    x = x_ref[...]
    xl_ref[...] = jnp.dot(x, wl_ref[...], preferred_element_type=F32) + bl_ref[...]
    xr_ref[...] = jnp.dot(x, wr_ref[...], preferred_element_type=F32) + br_ref[...]


def _prep(x, wl, bl, wr, br):
    return pl.pallas_call(
        _prep_body,
        grid=(_NGRID,),
        in_specs=[
            pl.BlockSpec((_RBLK, D), lambda i: (i, 0)),
            pl.BlockSpec((D, D), lambda i: (0, 0)),
            pl.BlockSpec((1, D), lambda i: (0, 0)),
            pl.BlockSpec((D, D), lambda i: (0, 0)),
            pl.BlockSpec((1, D), lambda i: (0, 0)),
        ],
        out_specs=[
            pl.BlockSpec((_RBLK, D), lambda i: (i, 0)),
            pl.BlockSpec((_RBLK, D), lambda i: (i, 0)),
        ],
        out_shape=[
            jax.ShapeDtypeStruct((N, D), F32),
            jax.ShapeDtypeStruct((N, D), F32),
        ],
    )(x, wl, bl, wr, br)


def _combine_block(accs_ref, den_ref, xl_ref, xr_ref, gsum_ref, att_ref, b_ref):
    acc = accs_ref[0] + accs_ref[1]
    xl = xl_ref[...]
    xr = xr_ref[...]
    den_e = lax.dot_general(
        den_ref[...], jnp.ones((NW, 1), F32),
        (((0,), (0,)), ((), ())), preferred_element_type=F32,
    )
    mean_g = jnp.sum(gsum_ref[...], axis=0, keepdims=True) * (1.0 / E)
    e = xl + xr + mean_g
    e = jnp.maximum(e, 0.2 * e)
    logit = jnp.sum(e * att_ref[...], axis=1, keepdims=True)
    w = jnp.exp(logit)
    den = den_e + w + 1e-16
    num = acc + w * xl
    return jnp.maximum(num / den + b_ref[...], 0.0)


def _combine_body(accs_ref, den_ref, xl_ref, xr_ref, gsum_ref, att_ref, b_ref,
                  out_ref):
    out_ref[...] = _combine_block(accs_ref, den_ref, xl_ref, xr_ref, gsum_ref,
                                  att_ref, b_ref)


def _combine_prep_body(accs_ref, den_ref, xl_ref, xr_ref, gsum_ref, att_ref,
                       b_ref, wl_ref, bl_ref, wr_ref, br_ref, xl2_ref, xr2_ref):
    x = _combine_block(accs_ref, den_ref, xl_ref, xr_ref, gsum_ref, att_ref,
                       b_ref)
    xl2_ref[...] = jnp.dot(x, wl_ref[...], preferred_element_type=F32) + bl_ref[...]
    xr2_ref[...] = jnp.dot(x, wr_ref[...], preferred_element_type=F32) + br_ref[...]


def _mk_combine_specs():
    return [
        pl.BlockSpec((2, _RBLK, D), lambda i: (0, i, 0)),
        pl.BlockSpec((NW, _RBLK), lambda i: (0, i)),
        pl.BlockSpec((_RBLK, D), lambda i: (i, 0)),
        pl.BlockSpec((_RBLK, D), lambda i: (i, 0)),
        pl.BlockSpec((8, D), lambda i: (0, 0)),
        pl.BlockSpec((1, D), lambda i: (0, 0)),
        pl.BlockSpec((1, D), lambda i: (0, 0)),
    ]


def _combine(accs, den, xl, xr, gsum, att, b):
    return pl.pallas_call(
        _combine_body,
        grid=(_NGRID,),
        in_specs=_mk_combine_specs(),
        out_specs=pl.BlockSpec((_RBLK, D), lambda i: (i, 0)),
        out_shape=jax.ShapeDtypeStruct((N, D), F32),
    )(accs, den, xl, xr, gsum, att, b)


def _combine_prep(accs, den, xl, xr, gsum, att, b, wl, bl, wr, br):
    return pl.pallas_call(
        _combine_prep_body,
        grid=(_NGRID,),
        in_specs=_mk_combine_specs() + [
            pl.BlockSpec((D, D), lambda i: (0, 0)),
            pl.BlockSpec((1, D), lambda i: (0, 0)),
            pl.BlockSpec((D, D), lambda i: (0, 0)),
            pl.BlockSpec((1, D), lambda i: (0, 0)),
        ],
        out_specs=[
            pl.BlockSpec((_RBLK, D), lambda i: (i, 0)),
            pl.BlockSpec((_RBLK, D), lambda i: (i, 0)),
        ],
        out_shape=[
            jax.ShapeDtypeStruct((N, D), F32),
            jax.ShapeDtypeStruct((N, D), F32),
        ],
    )(accs, den, xl, xr, gsum, att, b, wl, bl, wr, br)


_SC_MESH = plsc.VectorSubcoreMesh(core_axis_name="c", subcore_axis_name="s")


@functools.partial(
    pl.kernel,
    out_type=(
        jax.ShapeDtypeStruct((NC, N, D), F32),
        jax.ShapeDtypeStruct((NW * N,), F32),
    ),
    mesh=_SC_MESH,
    compiler_params=pltpu.CompilerParams(needs_layout_passes=False),
    scratch_types=[
        pltpu.VMEM((KSB + 1, CH), jnp.int32),
        pltpu.VMEM((KSB + 1, CH), jnp.int32),
        pltpu.VMEM((CH, D), F32),
        pltpu.VMEM((CH, D), F32),
        pltpu.VMEM((CH, D), F32),
        pltpu.VMEM((CH, D), F32),
        pltpu.VMEM((CH, D), F32),
        pltpu.VMEM((CH, D), F32),
        pltpu.VMEM((768,), F32),
        pltpu.VMEM((N,), F32),
        pltpu.VMEM((D,), F32),
        pltpu.VMEM_SHARED((N, D), F32),
        pltpu.SemaphoreType.DMA,
        pltpu.SemaphoreType.DMA,
        pltpu.SemaphoreType.DMA,
        pltpu.SemaphoreType.DMA,
        pltpu.SemaphoreType.DMA,
        pltpu.SemaphoreType.DMA,
        pltpu.SemaphoreType.DMA,
        pltpu.SemaphoreType.DMA,
        pltpu.SemaphoreType.DMA,
    ],
)
def _edge_kernel(xl_hbm, xr_hbm, g_hbm, src2_hbm, dst2_hbm, att_hbm,
                 acc_out, den_out,
                 sidx_v, didx_v,
                 xl0, xl1, xr0, xr1, g0, g1, w_v, den_v, att_v,
                 acc_sh, semxl0, semxl1, semxr0, semxr1, semg0, semg1,
                 sems0, sems1, semi):
    c = lax.axis_index("c")
    s = lax.axis_index("s")
    wid = s * NC + c
    ebase = wid * EPW
    rbase = wid * NCHUNK

    zero16 = jnp.zeros((16,), F32)
    xl = (xl0, xl1)
    xr = (xr0, xr1)
    g = (g0, g1)
    semxl = (semxl0, semxl1)
    semxr = (semxr0, semxr1)
    semg = (semg0, semg1)
    sems = (sems0, sems1)

    def _zden(i, _):
        den_v[pl.ds(i * 16, 16)] = zero16
        return 0

    lax.fori_loop(0, NR, _zden, 0)

    def _zrow(i, _):
        for k in range(D // 16):
            xl0[i, pl.ds(k * 16, 16)] = zero16
        return 0

    lax.fori_loop(0, CH, _zrow, 0)

    nz = N // CH
    ntrips = (nz - s + NS - 1) // NS

    def _zchunk(jj, _):
        j = s + jj * NS
        pltpu.sync_copy(xl0, acc_sh.at[pl.ds(j * CH, CH)])
        return 0

    lax.fori_loop(0, ntrips, _zchunk, 0)

    pltpu.sync_copy(att_hbm, att_v)
    att_c = [att_v[pl.ds(k * 16, 16)] for k in range(D // 16)]
    lane = lax.iota(jnp.int32, 16)
    tail_mask = lane < 8

    plsc.subcore_barrier()

    def _stage_idx(sb):
        widx = wid * NSB + sb
        pltpu.sync_copy(src2_hbm.at[widx], sidx_v.at[pl.ds(0, KSB)])
        pltpu.sync_copy(dst2_hbm.at[widx], didx_v.at[pl.ds(0, KSB)])

    def _issue_gathers(sb, q, b):
        base = ebase + sb * SBLK + q * CH
        pltpu.async_copy(xl_hbm.at[sidx_v.at[q]], xl[b], semxl[b])
        pltpu.async_copy(xr_hbm.at[didx_v.at[q]], xr[b], semxr[b])
        pltpu.async_copy(g_hbm.at[pl.ds(base, CH)], g[b], semg[b])

    def _wait_gathers(sb, q, b):
        base = ebase + sb * SBLK + q * CH
        pltpu.make_async_copy(xl_hbm.at[sidx_v.at[q]], xl[b], semxl[b]).wait()
        pltpu.make_async_copy(xr_hbm.at[didx_v.at[q]], xr[b], semxr[b]).wait()
        pltpu.make_async_copy(g_hbm.at[pl.ds(base, CH)], g[b], semg[b]).wait()

    def _compute(q, b):
        @plsc.parallel_loop(0, CH, unroll=4)
        def _(i):
            xlc = [xl[b][i, pl.ds(k * 16, 16)] for k in range(D // 16)]
            acc = zero16
            for k in range(D // 16):
                v = xlc[k] + xr[b][i, pl.ds(k * 16, 16)] + g[b][i, pl.ds(k * 16, 16)]
                v = jnp.maximum(v, 0.2 * v)
                acc = acc + v * att_c[k]
            logit = jnp.sum(acc)
            w = jnp.exp(jnp.full((16,), logit, F32))
            for k in range(D // 16):
                xl[b][i, pl.ds(k * 16, 16)] = xlc[k] * w
            w_v[pl.ds(i * 16, 16)] = w

        qv = jnp.full((16,), q, jnp.int32)
        for kq in range(3):
            mask = None if kq < 2 else tail_mask
            didx = plsc.load_gather(didx_v, [qv, kq * 16 + lane], mask=mask)
            wv = plsc.load_gather(w_v, [(kq * 16 + lane) * 16], mask=mask)
            plsc.addupdate_scatter(den_v, [didx], wv, mask=mask)

        pltpu.async_copy(xl[b], acc_sh.at[didx_v.at[q]], sems[b], add=True)

    def _wait_scatter(b):
        pltpu.make_async_copy(xl[b], acc_sh.at[didx_v.at[0]], sems[b]).wait()

    def sb_body(sb, _):
        def pair_body(t, _):
            q0 = 2 * t
            _wait_gathers(sb, q0, 0)

            @pl.when(t > 0)
            def _():
                _wait_scatter(1)

            _issue_gathers(sb, q0 + 1, 1)
            _compute(q0, 0)

            _wait_gathers(sb, q0 + 1, 1)

            @pl.when(t < KPAIR - 1)
            def _():
                _wait_scatter(0)
                _issue_gathers(sb, q0 + 2, 0)

            _compute(q0 + 1, 1)
            return 0

        lax.fori_loop(0, KPAIR, pair_body, 0)

        @pl.when(sb + 1 < NSB)
        def _():
            _wait_scatter(0)
            _wait_scatter(1)
            _stage_idx(sb + 1)
            _issue_gathers(sb + 1, 0, 0)

        return 0

    _stage_idx(0)
    _issue_gathers(0, 0, 0)
    lax.fori_loop(0, NSB, sb_body, 0)

    _wait_scatter(0)
    _wait_scatter(1)

    pltpu.sync_copy(den_v, den_out.at[pl.ds(wid * N, N)])

    plsc.subcore_barrier()

    def _wchunk(jj, _):
        j = s + jj * NS
        pltpu.sync_copy(acc_sh.at[pl.ds(j * CH, CH)],
                        acc_out.at[c, pl.ds(j * CH, CH)])
        return 0

    lax.fori_loop(0, ntrips, _wchunk, 0)


def kernel(node_fts, edge_index, edge_attr, Wl1, bl1, Wr1, br1, We1, att1, b1,
           Wl2, bl2, Wr2, br2, We2, att2, b2):
    src = edge_index[0]
    dst = edge_index[1]
    bl1r = bl1.reshape(1, D)
    br1r = br1.reshape(1, D)
    bl2r = bl2.reshape(1, D)
    br2r = br2.reshape(1, D)
    att1r = att1.reshape(1, D)
    att2r = att2.reshape(1, D)
    b1r = b1.reshape(1, D)
    b2r = b2.reshape(1, D)

    g1, gsum1 = _edge_feats(edge_attr, We1)
    g2, gsum2 = _edge_feats(edge_attr, We2)

    src2 = src.reshape(NW * NSB, KSB, CH)
    dst2 = dst.reshape(NW * NSB, KSB, CH)

    xl1, xr1 = _prep(node_fts, Wl1, bl1r, Wr1, br1r)
    accs1, den1 = _edge_kernel(xl1, xr1, g1, src2, dst2, att1)
    xl2, xr2 = _combine_prep(accs1, den1.reshape(NW, N), xl1, xr1, gsum1,
                             att1r, b1r, Wl2, bl2r, Wr2, br2r)
    accs2, den2 = _edge_kernel(xl2, xr2, g2, src2, dst2, att2)
    return _combine(accs2, den2.reshape(NW, N), xl2, xr2, gsum2, att2r, b2r)
